# baseline (speedup 1.0000x reference)
import functools

import jax
import jax.numpy as jnp
from jax import lax
from jax.experimental import pallas as pl
from jax.experimental.pallas import tpu as pltpu
from jax.experimental.pallas import tpu_sc as plsc

N = 10000
E = 320000
F_IN = 128
H = 256
HH = 128
G = 64
C = 128

K = 64
CPT = 320
SCH = 32
SUP = CPT // SCH
EROWS = 16 * CPT
EPAD = EROWS * K
NRING = 4
RPT = 632
NACC = 16 * RPT
LAST = N - 15 * RPT
_ZCH = tuple((o, min(K, RPT - o)) for o in range(0, RPT, K))

_MESH = plsc.VectorSubcoreMesh(core_axis_name="c", subcore_axis_name="s")


@functools.partial(
    pl.kernel,
    mesh=_MESH,
    out_type=jax.ShapeDtypeStruct((N, HH), jnp.float32),
    scratch_types=[
        pltpu.VMEM_SHARED((NACC, HH), jnp.float32),
        pltpu.VMEM((SCH, K), jnp.int32),
        pltpu.VMEM((K, HH), jnp.float32),
    ],
)
def _deg(col_hbm, deg_hbm, dacc, cbig, ones_v):
    cid = lax.axis_index("c")
    sid = lax.axis_index("s")

    @pl.when(cid == 0)
    def _():
        def zfill(i, _):
            for t in range(HH // 16):
                ones_v[i, pl.ds(t * 16, 16)] = jnp.zeros((16,), jnp.float32)
            return 0
        lax.fori_loop(0, K, zfill, 0)
        for off, sz in _ZCH:
            pltpu.sync_copy(ones_v.at[pl.ds(0, sz)] if sz != K else ones_v,
                            dacc.at[pl.ds(sid * RPT + off, sz)])

        def ofill(i, _):
            for t in range(HH // 16):
                ones_v[i, pl.ds(t * 16, 16)] = jnp.ones((16,), jnp.float32)
            return 0
        lax.fori_loop(0, K, ofill, 0)
        plsc.subcore_barrier()

        def outer(s, _):
            pltpu.sync_copy(col_hbm.at[pl.ds(sid * CPT + s * SCH, SCH)], cbig)

            def body(j, _):
                pltpu.sync_copy(ones_v, dacc.at[cbig.at[j]], add=True)
                return 0
            lax.fori_loop(0, SCH, body, 0)
            return 0
        lax.fori_loop(0, SUP, outer, 0)
        plsc.subcore_barrier()

        @pl.when(sid < 15)
        def _():
            pltpu.sync_copy(dacc.at[pl.ds(sid * RPT, RPT)],
                            deg_hbm.at[pl.ds(sid * RPT, RPT)])

        @pl.when(sid == 15)
        def _():
            pltpu.sync_copy(dacc.at[pl.ds(15 * RPT, LAST)],
                            deg_hbm.at[pl.ds(15 * RPT, LAST)])


@functools.partial(
    pl.kernel,
    mesh=_MESH,
    out_type=jax.ShapeDtypeStruct((2 * N, HH), jnp.float32),
    scratch_types=[
        pltpu.VMEM_SHARED((NACC, HH), jnp.float32),
        pltpu.VMEM((SCH, K), jnp.int32),
        pltpu.VMEM((SCH, K), jnp.int32),
        pltpu.VMEM((K, HH), jnp.float32),
        pltpu.VMEM((K, HH), jnp.float32),
        pltpu.VMEM((K, HH), jnp.float32),
        pltpu.VMEM((K, HH), jnp.float32),
        pltpu.SemaphoreType.DMA,
        pltpu.SemaphoreType.DMA,
        pltpu.SemaphoreType.DMA,
        pltpu.SemaphoreType.DMA,
    ],
)
def _agg(y_hbm, row_hbm, col_hbm, seg_hbm,
         acc, rbig, cbig, rows0, rows1, rows2, rows3,
         sem0, sem1, sem2, sem3):
    rows = (rows0, rows1, rows2, rows3)
    sems = (sem0, sem1, sem2, sem3)
    cid = lax.axis_index("c")
    sid = lax.axis_index("s")

    def zfill(i, _):
        for t in range(HH // 16):
            rows0[i, pl.ds(t * 16, 16)] = jnp.zeros((16,), jnp.float32)
        return 0
    lax.fori_loop(0, K, zfill, 0)
    for off, sz in _ZCH:
        pltpu.sync_copy(rows0.at[pl.ds(0, sz)] if sz != K else rows0,
                        acc.at[pl.ds(sid * RPT + off, sz)])
    plsc.subcore_barrier()

    off_v = jnp.full((16,), 1, jnp.int32) * (cid * N)

    def outer(s, _):
        base = sid * CPT + s * SCH
        pltpu.sync_copy(row_hbm.at[pl.ds(base, SCH)], rbig)
        pltpu.sync_copy(col_hbm.at[pl.ds(base, SCH)], cbig)

        def adj(i, _):
            for t in range(K // 16):
                rbig[i, pl.ds(t * 16, 16)] = rbig[i, pl.ds(t * 16, 16)] + off_v
            return 0
        lax.fori_loop(0, SCH, adj, 0)

        for c in range(NRING):
            pltpu.async_copy(y_hbm.at[rbig.at[c]], rows[c], sems[c])
        for c in range(SCH):
            b = c % NRING
            pltpu.make_async_copy(y_hbm.at[rbig.at[c]], rows[b], sems[b]).wait()
            pltpu.sync_copy(rows[b], acc.at[cbig.at[c]], add=True)
            if c + NRING < SCH:
                pltpu.async_copy(y_hbm.at[rbig.at[c + NRING]], rows[b], sems[b])
        return 0
    lax.fori_loop(0, SUP, outer, 0)

    plsc.subcore_barrier()

    @pl.when(sid < 15)
    def _():
        pltpu.sync_copy(acc.at[pl.ds(sid * RPT, RPT)],
                        seg_hbm.at[pl.ds(cid * N + sid * RPT, RPT)])

    @pl.when(sid == 15)
    def _():
        pltpu.sync_copy(acc.at[pl.ds(15 * RPT, LAST)],
                        seg_hbm.at[pl.ds(cid * N + 15 * RPT, LAST)])


NB = 400
NSTEPS = N // NB

_DOT = dict(preferred_element_type=jnp.float32,
            precision=jax.lax.Precision.HIGHEST)


def _dis_of(deg_blk):
    return lax.rsqrt(deg_blk[:, 0:1] + 1.0)


def _tc1_body(x_ref, w_ref, deg_ref, out_ref):
    dis = _dis_of(deg_ref[...])
    y = jnp.dot(x_ref[...], w_ref[...], **_DOT) * dis
    out_ref[0] = y[:, :HH]
    out_ref[1] = y[:, HH:]


def _tc1(x, W1, deg16):
    return pl.pallas_call(
        _tc1_body,
        grid=(NSTEPS,),
        in_specs=[
            pl.BlockSpec((NB, F_IN), lambda i: (i, 0)),
            pl.BlockSpec((F_IN, H), lambda i: (0, 0)),
            pl.BlockSpec((NB, HH), lambda i: (i, 0)),
        ],
        out_specs=pl.BlockSpec((2, NB, HH), lambda i: (0, i, 0)),
        out_shape=jax.ShapeDtypeStruct((2, N, HH), jnp.float32),
    )(x, W1, deg16)


def _tcmid_body(seg_ref, y_ref, deg_ref, w_ref, b_ref, out_ref):
    dis = _dis_of(deg_ref[...])
    conv = jnp.concatenate(
        [seg_ref[0] + y_ref[0], seg_ref[1] + y_ref[1]], axis=1)
    h = jnp.maximum(dis * conv + b_ref[...], 0.0)
    y = jnp.dot(h, w_ref[...], **_DOT) * dis
    out_ref[0] = y[:, :HH]
    out_ref[1] = y[:, HH:]


def _tcmid(seg, ytab, deg16, b_prev, W_next):
    return pl.pallas_call(
        _tcmid_body,
        grid=(NSTEPS,),
        in_specs=[
            pl.BlockSpec((2, NB, HH), lambda i: (0, i, 0)),
            pl.BlockSpec((2, NB, HH), lambda i: (0, i, 0)),
            pl.BlockSpec((NB, HH), lambda i: (i, 0)),
            pl.BlockSpec((H, H), lambda i: (0, 0)),
            pl.BlockSpec((1, H), lambda i: (0, 0)),
        ],
        out_specs=pl.BlockSpec((2, NB, HH), lambda i: (0, i, 0)),
        out_shape=jax.ShapeDtypeStruct((2, N, HH), jnp.float32),
    )(seg, ytab, deg16, W_next, b_prev)


def _tcfinal_body(seg_ref, y_ref, deg_ref, b_ref, batch_ref, wl_ref, bl_ref,
                  out_ref, sums_ref, cnt_ref):
    i = pl.program_id(0)

    @pl.when(i == 0)
    def _():
        sums_ref[...] = jnp.zeros_like(sums_ref)
        cnt_ref[...] = jnp.zeros_like(cnt_ref)

    dis = _dis_of(deg_ref[...])
    conv = jnp.concatenate(
        [seg_ref[0] + y_ref[0], seg_ref[1] + y_ref[1]], axis=1)
    h = jnp.maximum(dis * conv + b_ref[...], 0.0)

    bt = batch_ref[0, 0, :]
    mask = (bt[:, None] == lax.broadcasted_iota(jnp.int32, (NB, G), 1)
            ).astype(jnp.float32)
    sums_ref[...] += lax.dot_general(mask, h, (((0,), (0,)), ((), ())), **_DOT)
    cnt_ref[0, :] += jnp.sum(mask, axis=0)

    @pl.when(i == NSTEPS - 1)
    def _():
        pooled = sums_ref[...] / jnp.maximum(cnt_ref[0, :], 1.0)[:, None]
        out_ref[...] = jnp.dot(pooled, wl_ref[...], **_DOT) + bl_ref[...]


def _tcfinal(seg, ytab, deg16, b3, batch3, Wl, bl):
    return pl.pallas_call(
        _tcfinal_body,
        grid=(NSTEPS,),
        in_specs=[
            pl.BlockSpec((2, NB, HH), lambda i: (0, i, 0)),
            pl.BlockSpec((2, NB, HH), lambda i: (0, i, 0)),
            pl.BlockSpec((NB, HH), lambda i: (i, 0)),
            pl.BlockSpec((1, H), lambda i: (0, 0)),
            pl.BlockSpec((1, 1, NB), lambda i: (i, 0, 0)),
            pl.BlockSpec((H, C), lambda i: (0, 0)),
            pl.BlockSpec((1, C), lambda i: (0, 0)),
        ],
        out_specs=pl.BlockSpec((G, C), lambda i: (0, 0)),
        out_shape=jax.ShapeDtypeStruct((G, C), jnp.float32),
        scratch_shapes=[
            pltpu.VMEM((G, H), jnp.float32),
            pltpu.VMEM((1, G), jnp.float32),
        ],
    )(seg, ytab, deg16, b3, batch3, Wl, bl)


def kernel(x, edge_index, batch, W1, b1, W2, b2, W3, b3, Wl, bl):
    pad = EPAD - E
    row2 = jnp.concatenate(
        [edge_index[0], jnp.zeros((pad,), jnp.int32)]).reshape(EROWS, K)
    col2 = jnp.concatenate(
        [edge_index[1], jnp.full((pad,), N, jnp.int32)]).reshape(EROWS, K)
    batch3 = batch.reshape(NSTEPS, 1, NB)
    b1r = b1.reshape(1, H)
    b2r = b2.reshape(1, H)
    b3r = b3.reshape(1, H)
    blr = bl.reshape(1, C)

    deg16 = _deg(col2)

    ytab1 = _tc1(x, W1, deg16)
    seg1 = _agg(ytab1.reshape(2 * N, HH), row2, col2).reshape(2, N, HH)
    ytab2 = _tcmid(seg1, ytab1, deg16, b1r, W2)
    seg2 = _agg(ytab2.reshape(2 * N, HH), row2, col2).reshape(2, N, HH)
    ytab3 = _tcmid(seg2, ytab2, deg16, b2r, W3)
    seg3 = _agg(ytab3.reshape(2 * N, HH), row2, col2).reshape(2, N, HH)
    return _tcfinal(seg3, ytab3, deg16, b3r, batch3, Wl, blr)


# speedup vs baseline: 1.0420x; 1.0420x over previous
import functools

import jax
import jax.numpy as jnp
from jax import lax
from jax.experimental import pallas as pl
from jax.experimental.pallas import tpu as pltpu
from jax.experimental.pallas import tpu_sc as plsc

N = 10000
E = 320000
F_IN = 128
H = 256
HH = 128
G = 64
C = 128

K = 128
CPT = 160
SCH = 32
SUP = CPT // SCH
EROWS = 16 * CPT
EPAD = EROWS * K
RPT = 632
NACC = 16 * RPT
LAST = N - 15 * RPT

_MESH = plsc.VectorSubcoreMesh(core_axis_name="c", subcore_axis_name="s")


@functools.partial(
    pl.kernel,
    mesh=_MESH,
    out_type=jax.ShapeDtypeStruct((2, N, HH), jnp.float32),
    scratch_types=[
        pltpu.VMEM_SHARED((NACC, HH), jnp.float32),
        pltpu.VMEM((SCH, K), jnp.int32),
        pltpu.VMEM((K, HH), jnp.float32),
    ],
)
def _deg(col_hbm, deg_hbm, dacc, cbig, ones_v):
    cid = lax.axis_index("c")
    sid = lax.axis_index("s")

    def zfill(i, _):
        for t in range(HH // 16):
            ones_v[i, pl.ds(t * 16, 16)] = jnp.zeros((16,), jnp.float32)
        return 0
    lax.fori_loop(0, K, zfill, 0)
    for off, sz in ((0, K), (K, K), (2 * K, K), (3 * K, K), (4 * K, RPT - 4 * K)):
        pltpu.sync_copy(ones_v.at[pl.ds(0, sz)] if sz != K else ones_v,
                        dacc.at[pl.ds(sid * RPT + off, sz)])

    def ofill(i, _):
        for t in range(HH // 16):
            ones_v[i, pl.ds(t * 16, 16)] = jnp.ones((16,), jnp.float32)
        return 0
    lax.fori_loop(0, K, ofill, 0)
    plsc.subcore_barrier()

    def outer(s, _):
        pltpu.sync_copy(col_hbm.at[pl.ds(sid * CPT + s * SCH, SCH)], cbig)

        def body(j, _):
            pltpu.sync_copy(ones_v, dacc.at[cbig.at[j]], add=True)
            return 0
        lax.fori_loop(cid * (SCH // 2), (cid + 1) * (SCH // 2), body, 0)
        return 0
    lax.fori_loop(0, SUP, outer, 0)
    plsc.subcore_barrier()

    @pl.when(sid < 15)
    def _():
        pltpu.sync_copy(dacc.at[pl.ds(sid * RPT, RPT)],
                        deg_hbm.at[cid, pl.ds(sid * RPT, RPT)])

    @pl.when(sid == 15)
    def _():
        pltpu.sync_copy(dacc.at[pl.ds(15 * RPT, LAST)],
                        deg_hbm.at[cid, pl.ds(15 * RPT, LAST)])


@functools.partial(
    pl.kernel,
    mesh=_MESH,
    out_type=jax.ShapeDtypeStruct((2 * N, HH), jnp.float32),
    scratch_types=[
        pltpu.VMEM_SHARED((NACC, HH), jnp.float32),
        pltpu.VMEM((SCH, K), jnp.int32),
        pltpu.VMEM((SCH, K), jnp.int32),
        pltpu.VMEM((K, HH), jnp.float32),
        pltpu.VMEM((K, HH), jnp.float32),
        pltpu.SemaphoreType.DMA,
        pltpu.SemaphoreType.DMA,
    ],
)
def _agg(y_hbm, row_hbm, col_hbm, seg_hbm,
         acc, rbig, cbig, rows0, rows1, sem0, sem1):
    cid = lax.axis_index("c")
    sid = lax.axis_index("s")

    def zfill(i, _):
        for t in range(HH // 16):
            rows0[i, pl.ds(t * 16, 16)] = jnp.zeros((16,), jnp.float32)
        return 0
    lax.fori_loop(0, K, zfill, 0)
    for off, sz in ((0, K), (K, K), (2 * K, K), (3 * K, K), (4 * K, RPT - 4 * K)):
        pltpu.sync_copy(rows0.at[pl.ds(0, sz)] if sz != K else rows0,
                        acc.at[pl.ds(sid * RPT + off, sz)])
    plsc.subcore_barrier()

    off_v = jnp.full((16,), 1, jnp.int32) * (cid * N)

    def outer(s, _):
        base = sid * CPT + s * SCH
        pltpu.sync_copy(row_hbm.at[pl.ds(base, SCH)], rbig)
        pltpu.sync_copy(col_hbm.at[pl.ds(base, SCH)], cbig)

        def adj(i, _):
            for t in range(K // 16):
                rbig[i, pl.ds(t * 16, 16)] = rbig[i, pl.ds(t * 16, 16)] + off_v
            return 0
        lax.fori_loop(0, SCH, adj, 0)

        pltpu.async_copy(y_hbm.at[rbig.at[0]], rows0, sem0)

        def body(i, _):
            a = 2 * i
            b = a + 1
            g1 = pltpu.async_copy(y_hbm.at[rbig.at[b]], rows1, sem1)
            pltpu.make_async_copy(y_hbm.at[rbig.at[a]], rows0, sem0).wait()
            pltpu.sync_copy(rows0, acc.at[cbig.at[a]], add=True)

            @pl.when(i < SCH // 2 - 1)
            def _():
                pltpu.async_copy(y_hbm.at[rbig.at[a + 2]], rows0, sem0)

            g1.wait()
            pltpu.sync_copy(rows1, acc.at[cbig.at[b]], add=True)
            return 0
        lax.fori_loop(0, SCH // 2, body, 0)
        return 0
    lax.fori_loop(0, SUP, outer, 0)

    plsc.subcore_barrier()

    @pl.when(sid < 15)
    def _():
        pltpu.sync_copy(acc.at[pl.ds(sid * RPT, RPT)],
                        seg_hbm.at[pl.ds(cid * N + sid * RPT, RPT)])

    @pl.when(sid == 15)
    def _():
        pltpu.sync_copy(acc.at[pl.ds(15 * RPT, LAST)],
                        seg_hbm.at[pl.ds(cid * N + 15 * RPT, LAST)])


NB = 400
NSTEPS = N // NB

_DOT = dict(preferred_element_type=jnp.float32,
            precision=jax.lax.Precision.HIGHEST)


def _dis_of(deg_blk):
    return lax.rsqrt(deg_blk[0, :, 0:1] + deg_blk[1, :, 0:1] + 1.0)


def _tc1_body(x_ref, w_ref, deg_ref, out_ref):
    dis = _dis_of(deg_ref[...])
    y = jnp.dot(x_ref[...], w_ref[...], **_DOT) * dis
    out_ref[0] = y[:, :HH]
    out_ref[1] = y[:, HH:]


def _tc1(x, W1, deg16):
    return pl.pallas_call(
        _tc1_body,
        grid=(NSTEPS,),
        in_specs=[
            pl.BlockSpec((NB, F_IN), lambda i: (i, 0)),
            pl.BlockSpec((F_IN, H), lambda i: (0, 0)),
            pl.BlockSpec((2, NB, HH), lambda i: (0, i, 0)),
        ],
        out_specs=pl.BlockSpec((2, NB, HH), lambda i: (0, i, 0)),
        out_shape=jax.ShapeDtypeStruct((2, N, HH), jnp.float32),
    )(x, W1, deg16)


def _tcmid_body(seg_ref, y_ref, deg_ref, w_ref, b_ref, out_ref):
    dis = _dis_of(deg_ref[...])
    conv = jnp.concatenate(
        [seg_ref[0] + y_ref[0], seg_ref[1] + y_ref[1]], axis=1)
    h = jnp.maximum(dis * conv + b_ref[...], 0.0)
    y = jnp.dot(h, w_ref[...], **_DOT) * dis
    out_ref[0] = y[:, :HH]
    out_ref[1] = y[:, HH:]


def _tcmid(seg, ytab, deg16, b_prev, W_next):
    return pl.pallas_call(
        _tcmid_body,
        grid=(NSTEPS,),
        in_specs=[
            pl.BlockSpec((2, NB, HH), lambda i: (0, i, 0)),
            pl.BlockSpec((2, NB, HH), lambda i: (0, i, 0)),
            pl.BlockSpec((2, NB, HH), lambda i: (0, i, 0)),
            pl.BlockSpec((H, H), lambda i: (0, 0)),
            pl.BlockSpec((1, H), lambda i: (0, 0)),
        ],
        out_specs=pl.BlockSpec((2, NB, HH), lambda i: (0, i, 0)),
        out_shape=jax.ShapeDtypeStruct((2, N, HH), jnp.float32),
    )(seg, ytab, deg16, W_next, b_prev)


def _tcfinal_body(seg_ref, y_ref, deg_ref, b_ref, batch_ref, wl_ref, bl_ref,
                  out_ref, sums_ref, cnt_ref):
    i = pl.program_id(0)

    @pl.when(i == 0)
    def _():
        sums_ref[...] = jnp.zeros_like(sums_ref)
        cnt_ref[...] = jnp.zeros_like(cnt_ref)

    dis = _dis_of(deg_ref[...])
    conv = jnp.concatenate(
        [seg_ref[0] + y_ref[0], seg_ref[1] + y_ref[1]], axis=1)
    h = jnp.maximum(dis * conv + b_ref[...], 0.0)

    bt = batch_ref[0, 0, :]
    mask = (bt[:, None] == lax.broadcasted_iota(jnp.int32, (NB, G), 1)
            ).astype(jnp.float32)
    sums_ref[...] += lax.dot_general(mask, h, (((0,), (0,)), ((), ())), **_DOT)
    cnt_ref[0, :] += jnp.sum(mask, axis=0)

    @pl.when(i == NSTEPS - 1)
    def _():
        pooled = sums_ref[...] / jnp.maximum(cnt_ref[0, :], 1.0)[:, None]
        out_ref[...] = jnp.dot(pooled, wl_ref[...], **_DOT) + bl_ref[...]


def _tcfinal(seg, ytab, deg16, b3, batch3, Wl, bl):
    return pl.pallas_call(
        _tcfinal_body,
        grid=(NSTEPS,),
        in_specs=[
            pl.BlockSpec((2, NB, HH), lambda i: (0, i, 0)),
            pl.BlockSpec((2, NB, HH), lambda i: (0, i, 0)),
            pl.BlockSpec((2, NB, HH), lambda i: (0, i, 0)),
            pl.BlockSpec((1, H), lambda i: (0, 0)),
            pl.BlockSpec((1, 1, NB), lambda i: (i, 0, 0)),
            pl.BlockSpec((H, C), lambda i: (0, 0)),
            pl.BlockSpec((1, C), lambda i: (0, 0)),
        ],
        out_specs=pl.BlockSpec((G, C), lambda i: (0, 0)),
        out_shape=jax.ShapeDtypeStruct((G, C), jnp.float32),
        scratch_shapes=[
            pltpu.VMEM((G, H), jnp.float32),
            pltpu.VMEM((1, G), jnp.float32),
        ],
    )(seg, ytab, deg16, b3, batch3, Wl, bl)


def kernel(x, edge_index, batch, W1, b1, W2, b2, W3, b3, Wl, bl):
    pad = EPAD - E
    row2 = jnp.concatenate(
        [edge_index[0], jnp.zeros((pad,), jnp.int32)]).reshape(EROWS, K)
    col2 = jnp.concatenate(
        [edge_index[1], jnp.full((pad,), N, jnp.int32)]).reshape(EROWS, K)
    batch3 = batch.reshape(NSTEPS, 1, NB)
    b1r = b1.reshape(1, H)
    b2r = b2.reshape(1, H)
    b3r = b3.reshape(1, H)
    blr = bl.reshape(1, C)

    deg16 = _deg(col2)

    ytab1 = _tc1(x, W1, deg16)
    seg1 = _agg(ytab1.reshape(2 * N, HH), row2, col2).reshape(2, N, HH)
    ytab2 = _tcmid(seg1, ytab1, deg16, b1r, W2)
    seg2 = _agg(ytab2.reshape(2 * N, HH), row2, col2).reshape(2, N, HH)
    ytab3 = _tcmid(seg2, ytab2, deg16, b2r, W3)
    seg3 = _agg(ytab3.reshape(2 * N, HH), row2, col2).reshape(2, N, HH)
    return _tcfinal(seg3, ytab3, deg16, b3r, batch3, Wl, blr)
